# baseline (speedup 1.0000x reference)
import functools
import math

import numpy as np
import jax
import jax.numpy as jnp
from jax import lax
from jax.experimental import pallas as pl
from jax.experimental.pallas import tpu as pltpu
from jax.experimental.pallas import tpu_sc as plsc

D_MODEL = 64
_SCALE = 8.0
_L = 16


@jax.jit
def _run(xt, table, pe):
    S, NW, BB = xt.shape
    V, D = table.shape
    n_groups = S // 4

    mesh = plsc.VectorSubcoreMesh(core_axis_name="c", subcore_axis_name="s")

    @functools.partial(
        pl.kernel,
        out_type=jax.ShapeDtypeStruct((S, D // 8, NW, 8, BB), jnp.float32),
        mesh=mesh,
        scratch_types=[
            pltpu.VMEM((S, BB), jnp.int32),
            pltpu.VMEM((BB, D), jnp.float32),
            pltpu.VMEM((BB, D), jnp.float32),
            pltpu.VMEM((BB, D), jnp.float32),
            pltpu.VMEM((BB, D), jnp.float32),
            pltpu.VMEM((D // 8, 8, BB), jnp.float32),
            pltpu.VMEM((D // 8, 8, BB), jnp.float32),
            pltpu.VMEM((S, D), jnp.float32),
        ]
        + [pltpu.SemaphoreType.DMA] * 7,
        compiler_params=pltpu.CompilerParams(
            use_tc_tiling_on_sc=False, needs_layout_passes=False
        ),
    )
    def sc_kernel(xt_hbm, table_hbm, pe_hbm, out_hbm, idx_all, b0, b1, b2, b3,
                  t0, t1, pe_v, isem, g0, g1, g2, g3, o0, o1):
        bufs = (b0, b1, b2, b3)
        tbufs = (t0, t1)
        gsem = (g0, g1, g2, g3)
        osem = (o0, o1)
        bt = lax.axis_index("s") * 2 + lax.axis_index("c")

        pltpu.sync_copy(pe_hbm, pe_v)
        pltpu.async_copy(xt_hbm.at[:, bt], idx_all, isem).wait()

        def gather_start(p, b):
            pltpu.async_copy(table_hbm.at[idx_all.at[p]], bufs[b], gsem[b])

        def gather_wait(p, b):
            pltpu.make_async_copy(
                table_hbm.at[idx_all.at[p]], bufs[b], gsem[b]
            ).wait()

        def out_start(p, b):
            pltpu.async_copy(tbufs[b % 2], out_hbm.at[p, :, bt], osem[b % 2])

        def out_wait(b):
            pltpu.make_async_copy(
                tbufs[b % 2], out_hbm.at[0, :, bt], osem[b % 2]
            ).wait()

        for b in range(4):
            gather_start(b, b)

        rows = [lax.iota(jnp.int32, 16) + (16 * i) for i in range(8)]

        def group_body(g, carry):
            for b in range(4):
                p = 4 * g + b
                gather_wait(p, b)
                if b < 2:

                    @pl.when(g > 0)
                    def _():
                        out_wait(b)

                else:
                    out_wait(b)

                def d_body(d, dc):
                    dt = d // 8
                    dr = d % 8
                    col = jnp.full((_L,), d, jnp.int32)
                    pev = plsc.load_gather(
                        pe_v, [jnp.full((_L,), p, jnp.int32), col]
                    )
                    for i in range(8):
                        vals = plsc.load_gather(bufs[b], [rows[i], col])
                        tbufs[b % 2][dt, dr, pl.ds(16 * i, _L)] = (
                            vals * _SCALE + pev
                        )
                    return dc

                lax.fori_loop(0, D, d_body, 0, unroll=2)
                out_start(p, b)

                @pl.when(g + 1 < n_groups)
                def _():
                    gather_start(p + 4, b)

            return carry

        lax.fori_loop(0, n_groups, group_body, 0)
        out_wait(0)
        out_wait(1)

    return sc_kernel(xt, table, pe)


def kernel(x, table):
    B, S = x.shape
    V, D = table.shape
    NW = 32
    BB = B // NW

    pe = np.zeros((S, D_MODEL), dtype=np.float32)
    pos = np.arange(S, dtype=np.float32)[:, None]
    div_term = np.exp(
        np.arange(0, D_MODEL, 2, dtype=np.float32) * (-math.log(10000.0) / D_MODEL)
    )
    pe[:, 0::2] = np.sin(pos * div_term)
    pe[:, 1::2] = np.cos(pos * div_term)

    xt = x.T.reshape(S, NW, BB)
    out5 = _run(xt, table, jnp.asarray(pe))
    return out5.transpose(2, 4, 0, 1, 3).reshape(B, S, D)


# speedup vs baseline: 2.1984x; 2.1984x over previous
import functools
import math

import numpy as np
import jax
import jax.numpy as jnp
from jax import lax
from jax.experimental import pallas as pl
from jax.experimental.pallas import tpu as pltpu
from jax.experimental.pallas import tpu_sc as plsc

D_MODEL = 64
_SCALE = 8.0
_L = 16


@jax.jit
def _run(xt, table, pe):
    S, NW, BB = xt.shape
    V, D = table.shape
    n_groups = S // 4

    mesh = plsc.VectorSubcoreMesh(core_axis_name="c", subcore_axis_name="s")

    @functools.partial(
        pl.kernel,
        out_type=jax.ShapeDtypeStruct((S, D // 8, NW, 8, BB), jnp.float32),
        mesh=mesh,
        scratch_types=[
            pltpu.VMEM((S, BB), jnp.int32),
            pltpu.VMEM((BB, D), jnp.float32),
            pltpu.VMEM((BB, D), jnp.float32),
            pltpu.VMEM((BB, D), jnp.float32),
            pltpu.VMEM((BB, D), jnp.float32),
            pltpu.VMEM((D // 8, 8, BB + 1), jnp.float32),
            pltpu.VMEM((D // 8, 8, BB + 1), jnp.float32),
            pltpu.VMEM((S, D), jnp.float32),
        ]
        + [pltpu.SemaphoreType.DMA] * 7,
        compiler_params=pltpu.CompilerParams(
            use_tc_tiling_on_sc=False, needs_layout_passes=False
        ),
    )
    def sc_kernel(xt_hbm, table_hbm, pe_hbm, out_hbm, idx_all, b0, b1, b2, b3,
                  t0, t1, pe_v, isem, g0, g1, g2, g3, o0, o1):
        bufs = (b0, b1, b2, b3)
        tbufs = (t0, t1)
        gsem = (g0, g1, g2, g3)
        osem = (o0, o1)
        bt = lax.axis_index("s") * 2 + lax.axis_index("c")

        pltpu.sync_copy(pe_hbm, pe_v)
        pltpu.async_copy(xt_hbm.at[:, bt], idx_all, isem).wait()

        def gather_start(p, b):
            pltpu.async_copy(table_hbm.at[idx_all.at[p]], bufs[b], gsem[b])

        def gather_wait(p, b):
            pltpu.make_async_copy(
                table_hbm.at[idx_all.at[p]], bufs[b], gsem[b]
            ).wait()

        def out_start(p, b):
            pltpu.async_copy(
                tbufs[b % 2].at[:, :, pl.ds(0, BB)],
                out_hbm.at[p, :, bt],
                osem[b % 2],
            )

        def out_wait(b):
            pltpu.make_async_copy(
                tbufs[b % 2].at[:, :, pl.ds(0, BB)],
                out_hbm.at[0, :, bt],
                osem[b % 2],
            ).wait()

        for b in range(4):
            gather_start(b, b)

        lane = lax.iota(jnp.int32, 16)
        dts = [(lane + 16 * j) // 8 for j in range(D // _L)]
        drs = [lane % 8 for _ in range(D // _L)]

        def group_body(g, carry):
            for b in range(4):
                p = 4 * g + b
                gather_wait(p, b)
                if b < 2:

                    @pl.when(g > 0)
                    def _():
                        out_wait(b)

                else:
                    out_wait(b)

                pe4 = [pe_v[p, pl.ds(16 * j, _L)] for j in range(D // _L)]

                def i_body(i, ic):
                    ci = jnp.full((_L,), i, jnp.int32)
                    for j in range(D // _L):
                        v = bufs[b][i, pl.ds(16 * j, _L)] * _SCALE + pe4[j]
                        plsc.store_scatter(
                            tbufs[b % 2], [dts[j], drs[j], ci], v
                        )
                    return ic

                lax.fori_loop(0, BB, i_body, 0, unroll=2)
                out_start(p, b)

                @pl.when(g + 1 < n_groups)
                def _():
                    gather_start(p + 4, b)

            return carry

        lax.fori_loop(0, n_groups, group_body, 0)
        out_wait(0)
        out_wait(1)

    return sc_kernel(xt, table, pe)


def kernel(x, table):
    B, S = x.shape
    V, D = table.shape
    NW = 32
    BB = B // NW

    pe = np.zeros((S, D_MODEL), dtype=np.float32)
    pos = np.arange(S, dtype=np.float32)[:, None]
    div_term = np.exp(
        np.arange(0, D_MODEL, 2, dtype=np.float32) * (-math.log(10000.0) / D_MODEL)
    )
    pe[:, 0::2] = np.sin(pos * div_term)
    pe[:, 1::2] = np.cos(pos * div_term)

    xt = x.T.reshape(S, NW, BB)
    out5 = _run(xt, table, jnp.asarray(pe))
    return out5.transpose(2, 4, 0, 1, 3).reshape(B, S, D)
